# Initial kernel scaffold; baseline (speedup 1.0000x reference)
import jax
import jax.numpy as jnp
from jax.experimental import pallas as pl


def kernel(x, edge_index, batch, W1, a_src1, a_dst1, b1, W2, a_src2, a_dst2, b2, Wl1, bl1, Wl2, bl2):
    raise NotImplementedError("write your pallas kernel here")


# speedup vs baseline: 14.5070x; 14.5070x over previous
import functools

import jax
import jax.numpy as jnp
from jax import lax
from jax.experimental import pallas as pl
from jax.experimental.pallas import tpu as pltpu
from jax.experimental.pallas import tpu_sc as plsc

NN = 10000
NP = 10240
EE = 320000
CH = 128
NCHUNK = 79
EPT = CH * NCHUNK
NWORK = 32
EP = EPT * NWORK
DUMP = 10200
BLK = 1024


def _sc_agg_body(src_hbm, dst_hbm, asrc_hbm, adst_hbm, h_hbm,
                 num_out, den_out,
                 asrc_v, adst_v, sidx_v, didx_v, w_v, rows_v, sem,
                 num_sh, den_sh):
    cid = lax.axis_index("c")
    sid = lax.axis_index("s")

    zeros16 = jnp.zeros((16,), jnp.float32)

    def zrow(r, c):
        for cc in range(8):
            rows_v[r, pl.ds(cc * 16, 16)] = zeros16
        return c

    lax.fori_loop(0, CH, zrow, 0)
    for cc in range(8):
        w_v[pl.ds(cc * 16, 16)] = zeros16

    r0 = sid * 640
    for k in range(5):
        pltpu.sync_copy(rows_v, num_sh.at[pl.ds(r0 + k * 128, 128)])
        pltpu.sync_copy(w_v, den_sh.at[pl.ds(r0 + k * 128, 128)])

    pltpu.sync_copy(asrc_hbm, asrc_v)
    pltpu.sync_copy(adst_hbm, adst_v)
    plsc.subcore_barrier()

    wid = cid * 16 + sid
    ebase = wid * EPT

    def chunk(j, carry):
        eb = ebase + j * CH
        pltpu.sync_copy(src_hbm.at[pl.ds(eb, CH)], sidx_v)
        pltpu.sync_copy(dst_hbm.at[pl.ds(eb, CH)], didx_v)
        for t in range(8):
            si = sidx_v[pl.ds(t * 16, 16)]
            di = didx_v[pl.ds(t * 16, 16)]
            e = plsc.load_gather(asrc_v, [si]) + plsc.load_gather(adst_v, [di])
            e = jnp.maximum(e, 0.0) + 0.2 * jnp.minimum(e, 0.0)
            w_v[pl.ds(t * 16, 16)] = jnp.exp(e)
        pltpu.async_copy(h_hbm.at[sidx_v], rows_v, sem).wait()

        def scale(r, carry2):
            wr = plsc.load_gather(w_v, [jnp.zeros((16,), jnp.int32) + r])
            for cc in range(8):
                rows_v[r, pl.ds(cc * 16, 16)] = rows_v[r, pl.ds(cc * 16, 16)] * wr
            return carry2

        lax.fori_loop(0, CH, scale, 0)
        pltpu.sync_copy(rows_v, num_sh.at[didx_v], add=True)
        pltpu.sync_copy(w_v, den_sh.at[didx_v], add=True)
        return carry

    lax.fori_loop(0, NCHUNK, chunk, 0)
    plsc.subcore_barrier()

    for k in range(5):
        pltpu.sync_copy(num_sh.at[pl.ds(r0 + k * 128, 128)],
                        num_out.at[cid, pl.ds(r0 + k * 128, 128)])
        pltpu.sync_copy(den_sh.at[pl.ds(r0 + k * 128, 128)],
                        den_out.at[cid, pl.ds(r0 + k * 128, 128)])


@jax.jit
def _sc_agg(src, dst, a_s, a_d, h):
    mesh = plsc.VectorSubcoreMesh(core_axis_name="c", subcore_axis_name="s")
    f = pl.kernel(
        _sc_agg_body,
        mesh=mesh,
        compiler_params=pltpu.CompilerParams(needs_layout_passes=False),
        out_type=[
            jax.ShapeDtypeStruct((2, NP, 128), jnp.float32),
            jax.ShapeDtypeStruct((2, NP), jnp.float32),
        ],
        scratch_types=[
            pltpu.VMEM((NP,), jnp.float32),
            pltpu.VMEM((NP,), jnp.float32),
            pltpu.VMEM((CH,), jnp.int32),
            pltpu.VMEM((CH,), jnp.int32),
            pltpu.VMEM((CH,), jnp.float32),
            pltpu.VMEM((CH, 128), jnp.float32),
            pltpu.SemaphoreType.DMA,
            pltpu.VMEM_SHARED((NP, 128), jnp.float32),
            pltpu.VMEM_SHARED((NP,), jnp.float32),
        ],
    )
    return f(src, dst, a_s, a_d, h)


def _dense1_body(x_ref, w1a_ref, w1b_ref, a1sa_ref, a1sb_ref, a1da_ref,
                 a1db_ref, h1a_ref, h1b_ref, as1_ref, ad1_ref):
    x = x_ref[...]
    dn = (((1,), (1,)), ((), ()))
    ha = lax.dot_general(x, w1a_ref[...], dn, preferred_element_type=jnp.float32)
    hb = lax.dot_general(x, w1b_ref[...], dn, preferred_element_type=jnp.float32)
    h1a_ref[...] = ha
    h1b_ref[...] = hb
    dnv = (((1,), (0,)), ((), ()))
    as1_ref[...] = (lax.dot_general(ha, a1sa_ref[...], dnv, preferred_element_type=jnp.float32)
                    + lax.dot_general(hb, a1sb_ref[...], dnv, preferred_element_type=jnp.float32))
    ad1_ref[...] = (lax.dot_general(ha, a1da_ref[...], dnv, preferred_element_type=jnp.float32)
                    + lax.dot_general(hb, a1db_ref[...], dnv, preferred_element_type=jnp.float32))


def _rows_spec(w):
    return pl.BlockSpec((BLK, w), lambda i: (i, 0))


def _full_spec(r, c):
    return pl.BlockSpec((r, c), lambda i: (0, 0))


@jax.jit
def _dense1(x_p, W1a, W1b, a1sa, a1sb, a1da, a1db):
    return pl.pallas_call(
        _dense1_body,
        grid=(NP // BLK,),
        in_specs=[_rows_spec(128), _full_spec(128, 128), _full_spec(128, 128),
                  _full_spec(128, 1), _full_spec(128, 1),
                  _full_spec(128, 1), _full_spec(128, 1)],
        out_specs=[_rows_spec(128), _rows_spec(128), _rows_spec(1), _rows_spec(1)],
        out_shape=[jax.ShapeDtypeStruct((NP, 128), jnp.float32),
                   jax.ShapeDtypeStruct((NP, 128), jnp.float32),
                   jax.ShapeDtypeStruct((NP, 1), jnp.float32),
                   jax.ShapeDtypeStruct((NP, 1), jnp.float32)],
    )(x_p, W1a, W1b, a1sa, a1sb, a1da, a1db)


def _dense2_body(h1a_ref, h1b_ref, as1_ref, ad1_ref,
                 na0_ref, na1_ref, nb0_ref, nb1_ref, d0_ref, d1_ref,
                 b1a_ref, b1b_ref, w2a_ref, w2b_ref, a2s_ref, a2d_ref,
                 h2_ref, as2_ref, ad2_ref):
    e = as1_ref[...] + ad1_ref[...]
    wself = jnp.exp(jnp.maximum(e, 0.0) + 0.2 * jnp.minimum(e, 0.0))
    den = d0_ref[...] + d1_ref[...] + wself + 1e-16
    outa = jnp.maximum(
        (na0_ref[...] + na1_ref[...] + wself * h1a_ref[...]) / den + b1a_ref[...], 0.0)
    outb = jnp.maximum(
        (nb0_ref[...] + nb1_ref[...] + wself * h1b_ref[...]) / den + b1b_ref[...], 0.0)
    dn = (((1,), (1,)), ((), ()))
    h2 = (lax.dot_general(outa, w2a_ref[...], dn, preferred_element_type=jnp.float32)
          + lax.dot_general(outb, w2b_ref[...], dn, preferred_element_type=jnp.float32))
    h2_ref[...] = h2
    dnv = (((1,), (0,)), ((), ()))
    as2_ref[...] = lax.dot_general(h2, a2s_ref[...], dnv, preferred_element_type=jnp.float32)
    ad2_ref[...] = lax.dot_general(h2, a2d_ref[...], dnv, preferred_element_type=jnp.float32)


@jax.jit
def _dense2(h1a, h1b, as1, ad1, na0, na1, nb0, nb1, d0, d1,
            b1a, b1b, W2a, W2b, a2s, a2d):
    return pl.pallas_call(
        _dense2_body,
        grid=(NP // BLK,),
        in_specs=[_rows_spec(128), _rows_spec(128), _rows_spec(1), _rows_spec(1),
                  _rows_spec(128), _rows_spec(128), _rows_spec(128), _rows_spec(128),
                  _rows_spec(1), _rows_spec(1),
                  _full_spec(1, 128), _full_spec(1, 128),
                  _full_spec(128, 128), _full_spec(128, 128),
                  _full_spec(128, 1), _full_spec(128, 1)],
        out_specs=[_rows_spec(128), _rows_spec(1), _rows_spec(1)],
        out_shape=[jax.ShapeDtypeStruct((NP, 128), jnp.float32),
                   jax.ShapeDtypeStruct((NP, 1), jnp.float32),
                   jax.ShapeDtypeStruct((NP, 1), jnp.float32)],
    )(h1a, h1b, as1, ad1, na0, na1, nb0, nb1, d0, d1, b1a, b1b, W2a, W2b, a2s, a2d)


def _dense3_body(h2_ref, as2_ref, ad2_ref, n0_ref, n1_ref, d0_ref, d1_ref,
                 b2_ref, wl1_ref, bl1_ref, wl2_ref, bl2_ref, y_ref):
    e = as2_ref[...] + ad2_ref[...]
    wself = jnp.exp(jnp.maximum(e, 0.0) + 0.2 * jnp.minimum(e, 0.0))
    den = d0_ref[...] + d1_ref[...] + wself + 1e-16
    out2 = jnp.maximum(
        (n0_ref[...] + n1_ref[...] + wself * h2_ref[...]) / den + b2_ref[...], 0.0)
    dn = (((1,), (1,)), ((), ()))
    t = jnp.maximum(
        lax.dot_general(out2, wl1_ref[...], dn, preferred_element_type=jnp.float32)
        + bl1_ref[...], 0.0)
    y_ref[...] = (lax.dot_general(t, wl2_ref[...], dn, preferred_element_type=jnp.float32)
                  + bl2_ref[...])


@jax.jit
def _dense3(h2, as2, ad2, n0, n1, d0, d1, b2r, Wl1, bl1r, Wl2, bl2r):
    return pl.pallas_call(
        _dense3_body,
        grid=(NP // BLK,),
        in_specs=[_rows_spec(128), _rows_spec(1), _rows_spec(1),
                  _rows_spec(128), _rows_spec(128), _rows_spec(1), _rows_spec(1),
                  _full_spec(1, 128), _full_spec(256, 128), _full_spec(1, 256),
                  _full_spec(128, 256), _full_spec(1, 128)],
        out_specs=[_rows_spec(128)],
        out_shape=[jax.ShapeDtypeStruct((NP, 128), jnp.float32)],
    )(h2, as2, ad2, n0, n1, d0, d1, b2r, Wl1, bl1r, Wl2, bl2r)[0]


def kernel(x, edge_index, batch, W1, a_src1, a_dst1, b1,
           W2, a_src2, a_dst2, b2, Wl1, bl1, Wl2, bl2):
    del batch
    x_p = jnp.pad(x, ((0, NP - NN), (0, 0)))
    pad_e = EP - EE
    src = jnp.concatenate([edge_index[0], jnp.zeros((pad_e,), jnp.int32)])
    dst = jnp.concatenate([edge_index[1], jnp.full((pad_e,), DUMP, jnp.int32)])

    W1a, W1b = W1[:128], W1[128:]
    h1a, h1b, as1, ad1 = _dense1(
        x_p, W1a, W1b,
        a_src1[:128].reshape(128, 1), a_src1[128:].reshape(128, 1),
        a_dst1[:128].reshape(128, 1), a_dst1[128:].reshape(128, 1))

    as1v = as1.reshape(NP)
    ad1v = ad1.reshape(NP)
    numA, denA = _sc_agg(src, dst, as1v, ad1v, h1a)
    numB, _ = _sc_agg(src, dst, as1v, ad1v, h1b)

    h2, as2, ad2 = _dense2(
        h1a, h1b, as1, ad1,
        numA[0], numA[1], numB[0], numB[1],
        denA[0].reshape(NP, 1), denA[1].reshape(NP, 1),
        b1[:128].reshape(1, 128), b1[128:].reshape(1, 128),
        W2[:, :128], W2[:, 128:],
        a_src2.reshape(128, 1), a_dst2.reshape(128, 1))

    num2, den2 = _sc_agg(src, dst, as2.reshape(NP), ad2.reshape(NP), h2)

    y = _dense3(
        h2, as2, ad2, num2[0], num2[1],
        den2[0].reshape(NP, 1), den2[1].reshape(NP, 1),
        b2.reshape(1, 128), Wl1, bl1.reshape(1, 256), Wl2, bl2.reshape(1, 128))
    return y[:NN]
